# baseline (speedup 1.0000x reference)
import jax
import jax.numpy as jnp
from jax.experimental import pallas as pl
from jax.experimental.pallas import tpu as pltpu

_B = 8
_K = 1024
_V = 100000
_VC = 2048
_NC = (_V + _VC - 1) // _VC

try:
    _G = jax.random.gumbel(jax.random.key(42), (_B, _V), dtype=jnp.float32)
except Exception:
    _G = None


def _gumbel():
    if _G is not None:
        return _G
    return jax.random.gumbel(jax.random.key(42), (_B, _V), dtype=jnp.float32)


def _argmax_body(xt_ref, wt_ref, b_ref, g_ref, s_ref, sg_ref, bv_ref, bi_ref):
    i = pl.program_id(0)
    logits = jax.lax.dot_general(
        xt_ref[...], wt_ref[...],
        dimension_numbers=(((0,), (1,)), ((), ())),
        preferred_element_type=jnp.float32,
    )
    logits = logits + b_ref[...] + g_ref[...]
    cols = i * _VC + jax.lax.broadcasted_iota(jnp.int32, logits.shape, 1)
    logits = jnp.where(cols < _V, logits, -jnp.inf)
    m = jnp.max(logits, axis=1, keepdims=True)
    cand = jnp.min(
        jnp.where(logits == m, cols, jnp.int32(2**31 - 1)), axis=1, keepdims=True
    )

    @pl.when(i == 0)
    def _():
        bv_ref[...] = m
        bi_ref[...] = cand

    @pl.when(i > 0)
    def _():
        bv = bv_ref[...]
        upd = m > bv
        bv_ref[...] = jnp.where(upd, m, bv)
        bi_ref[...] = jnp.where(upd, cand, bi_ref[...])

    @pl.when(i == _NC - 1)
    def _():
        allcols = jax.lax.broadcasted_iota(jnp.int32, s_ref.shape, 1)
        oh = (allcols == bi_ref[...]).astype(jnp.float32)
        s_ref[...] = oh
        sg_ref[...] = oh


def kernel(x, W, b):
    xt = x.T
    wt = W.T
    b2 = b.reshape(1, _V)
    sample, sample_grad = pl.pallas_call(
        _argmax_body,
        grid=(_NC,),
        in_specs=[
            pl.BlockSpec((_K, _B), lambda i: (0, 0)),
            pl.BlockSpec((_VC, _K), lambda i: (i, 0)),
            pl.BlockSpec((1, _VC), lambda i: (0, i)),
            pl.BlockSpec((_B, _VC), lambda i: (0, i)),
        ],
        out_specs=[
            pl.BlockSpec((_B, _V), lambda i: (0, 0)),
            pl.BlockSpec((_B, _V), lambda i: (0, 0)),
        ],
        out_shape=[
            jax.ShapeDtypeStruct((_B, _V), jnp.float32),
            jax.ShapeDtypeStruct((_B, _V), jnp.float32),
        ],
        scratch_shapes=[
            pltpu.VMEM((_B, 1), jnp.float32),
            pltpu.VMEM((_B, 1), jnp.int32),
        ],
    )(xt, wt, b2, _gumbel())
    return (sample, sample_grad)


# speedup vs baseline: 1.0158x; 1.0158x over previous
import jax
import jax.numpy as jnp
from jax.experimental import pallas as pl
from jax.experimental.pallas import tpu as pltpu

_B = 8
_K = 1024
_V = 100000
_VC = 2048
_NC = (_V + _VC - 1) // _VC

try:
    _G = jax.random.gumbel(jax.random.key(42), (_B, _V), dtype=jnp.float32)
except Exception:
    _G = None


def _gumbel():
    if _G is not None:
        return _G
    return jax.random.gumbel(jax.random.key(42), (_B, _V), dtype=jnp.float32)


def _argmax_body(x_ref, wt_ref, b_ref, g_ref, s_ref, sg_ref, bv_ref, bi_ref):
    i = pl.program_id(0)
    logits = jax.lax.dot_general(
        x_ref[...], wt_ref[...],
        dimension_numbers=(((1,), (1,)), ((), ())),
        preferred_element_type=jnp.float32,
    )
    logits = logits + b_ref[...] + g_ref[...]
    cols = i * _VC + jax.lax.broadcasted_iota(jnp.int32, logits.shape, 1)
    logits = jnp.where(cols < _V, logits, -jnp.inf)
    m = jnp.max(logits, axis=1, keepdims=True)
    cand = jnp.min(
        jnp.where(logits == m, cols, jnp.int32(2**31 - 1)), axis=1, keepdims=True
    )

    @pl.when(i == 0)
    def _():
        bv_ref[...] = m
        bi_ref[...] = cand

    @pl.when(i > 0)
    def _():
        bv = bv_ref[...]
        upd = m > bv
        bv_ref[...] = jnp.where(upd, m, bv)
        bi_ref[...] = jnp.where(upd, cand, bi_ref[...])

    @pl.when(i == _NC - 1)
    def _():
        allcols = jax.lax.broadcasted_iota(jnp.int32, s_ref.shape, 1)
        oh = (allcols == bi_ref[...]).astype(jnp.float32)
        s_ref[...] = oh
        sg_ref[...] = oh


def kernel(x, W, b):
    wt = W.T
    b2 = b.reshape(1, _V)
    sample, sample_grad = pl.pallas_call(
        _argmax_body,
        grid=(_NC,),
        in_specs=[
            pl.BlockSpec((_B, _K), lambda i: (0, 0)),
            pl.BlockSpec((_VC, _K), lambda i: (i, 0)),
            pl.BlockSpec((1, _VC), lambda i: (0, i)),
            pl.BlockSpec((_B, _VC), lambda i: (0, i)),
        ],
        out_specs=[
            pl.BlockSpec((_B, _V), lambda i: (0, 0)),
            pl.BlockSpec((_B, _V), lambda i: (0, 0)),
        ],
        out_shape=[
            jax.ShapeDtypeStruct((_B, _V), jnp.float32),
            jax.ShapeDtypeStruct((_B, _V), jnp.float32),
        ],
        scratch_shapes=[
            pltpu.VMEM((_B, 1), jnp.float32),
            pltpu.VMEM((_B, 1), jnp.int32),
        ],
    )(x, wt, b2, _gumbel())
    return (sample, sample_grad)
